# Initial kernel scaffold; baseline (speedup 1.0000x reference)
import jax
import jax.numpy as jnp
from jax.experimental import pallas as pl


def kernel(cloud0, cloud1):
    raise NotImplementedError("write your pallas kernel here")


# speedup vs baseline: 97.9440x; 97.9440x over previous
import jax
import jax.numpy as jnp
from jax.experimental import pallas as pl
from jax.experimental.pallas import tpu as pltpu


def _grouping_body(p0_ref, f1_ref, g0_ref, g1_ref):
    R, W = g0_ref.shape
    lane = jax.lax.broadcasted_iota(jnp.int32, (R, W), 1)
    m = lane % 3
    x = p0_ref[:, 0:1]
    y = p0_ref[:, 1:2]
    z = p0_ref[:, 2:3]
    g0_ref[...] = jnp.where(m == 0, x, jnp.where(m == 1, y, z))
    g1_ref[...] = jnp.broadcast_to(f1_ref[0], (R, W))


def kernel(cloud0, cloud1):
    B0, C, N0 = cloud0.shape
    B1, _, N1 = cloud1.shape
    pts0 = jnp.transpose(cloud0, (0, 2, 1)).reshape(-1, C)
    pts1 = jnp.transpose(cloud1, (0, 2, 1)).reshape(-1, C)
    M0 = pts0.shape[0]
    W = N1 * C
    flat1 = pts1.reshape(B1, 1, W)

    R = 128
    grid = (M0 // R,)
    g0, g1 = pl.pallas_call(
        _grouping_body,
        grid=grid,
        in_specs=[
            pl.BlockSpec((R, C), lambda r: (r, 0)),
            pl.BlockSpec((1, 1, W), lambda r: (r * R // N0, 0, 0)),
        ],
        out_specs=[
            pl.BlockSpec((R, W), lambda r: (r, 0)),
            pl.BlockSpec((R, W), lambda r: (r, 0)),
        ],
        out_shape=[
            jax.ShapeDtypeStruct((M0, W), jnp.float32),
            jax.ShapeDtypeStruct((M0, W), jnp.float32),
        ],
    )(pts0, flat1)
    return (pts0, pts1, g0.reshape(M0, N1, C), g1.reshape(M0, N1, C))


# speedup vs baseline: 825.0351x; 8.4235x over previous
import jax
import jax.numpy as jnp
from jax.experimental import pallas as pl


def _grouping_body(q0_ref, c1_ref, g0_ref, g1_ref):
    C, R, W = g0_ref.shape
    g0_ref[...] = jnp.broadcast_to(q0_ref[...], (C, R, W))
    g1_ref[...] = jnp.broadcast_to(c1_ref[0][:, None, :], (C, R, W))


def kernel(cloud0, cloud1):
    B0, C, N0 = cloud0.shape
    B1, _, N1 = cloud1.shape
    M0, M1 = B0 * N0, B1 * N1
    pts0 = jnp.transpose(cloud0, (0, 2, 1)).reshape(M0, C)
    pts1 = jnp.transpose(cloud1, (0, 2, 1)).reshape(M1, C)
    q0 = jnp.transpose(cloud0, (1, 0, 2)).reshape(C, M0, 1)

    R = 256
    grid = (M0 // R,)
    g0t, g1t = pl.pallas_call(
        _grouping_body,
        grid=grid,
        in_specs=[
            pl.BlockSpec((C, R, 1), lambda r: (0, r, 0)),
            pl.BlockSpec((1, C, N1), lambda r: (r * R // N0, 0, 0)),
        ],
        out_specs=[
            pl.BlockSpec((C, R, N1), lambda r: (0, r, 0)),
            pl.BlockSpec((C, R, N1), lambda r: (0, r, 0)),
        ],
        out_shape=[
            jax.ShapeDtypeStruct((C, M0, N1), jnp.float32),
            jax.ShapeDtypeStruct((C, M0, N1), jnp.float32),
        ],
    )(q0, cloud1)
    return (
        pts0,
        pts1,
        jnp.transpose(g0t, (1, 2, 0)),
        jnp.transpose(g1t, (1, 2, 0)),
    )
